# Initial kernel scaffold; baseline (speedup 1.0000x reference)
import jax
import jax.numpy as jnp
from jax.experimental import pallas as pl


def kernel(item_ids, genre_ids, director_ids, writer_ids, item_table, genre_table, director_table, writer_table, W, b):
    raise NotImplementedError("write your pallas kernel here")


# speedup vs baseline: 6.6521x; 6.6521x over previous
import functools

import jax
import jax.numpy as jnp
from jax import lax
from jax.experimental import pallas as pl
from jax.experimental.pallas import tpu as pltpu
from jax.experimental.pallas import tpu_sc as plsc

_NC = 2
_NS = 16
_NW = _NC * _NS
_D = 64
_T = 64


def _sc_gather_pool(item_ids, gids, dids, wids, item_tab, g_tab, d_tab, w_tab):
    BL = item_ids.shape[0]
    tok_per_w = BL // _NW
    nblk = tok_per_w // _T

    mesh = plsc.VectorSubcoreMesh(core_axis_name="c", subcore_axis_name="s")

    @functools.partial(
        pl.kernel,
        out_type=[jax.ShapeDtypeStruct((BL, _D), jnp.float32) for _ in range(4)],
        mesh=mesh,
        compiler_params=pltpu.CompilerParams(use_tc_tiling_on_sc=False),
        scratch_types=dict(
            iidx=pltpu.VMEM((_T,), jnp.int32),
            gidx=pltpu.VMEM((5, _T), jnp.int32),
            didx=pltpu.VMEM((5, _T), jnp.int32),
            widx=pltpu.VMEM((5, _T), jnp.int32),
            ibuf=pltpu.VMEM((_T, _D), jnp.float32),
            gbufs=[pltpu.VMEM((_T, _D), jnp.float32) for _ in range(5)],
            dbufs=[pltpu.VMEM((_T, _D), jnp.float32) for _ in range(5)],
            wbufs=[pltpu.VMEM((_T, _D), jnp.float32) for _ in range(5)],
            gacc=pltpu.VMEM((_T, _D), jnp.float32),
            dacc=pltpu.VMEM((_T, _D), jnp.float32),
            wacc=pltpu.VMEM((_T, _D), jnp.float32),
            sem=pltpu.SemaphoreType.DMA,
        ),
    )
    def body(item_ids_h, gids_h, dids_h, wids_h, itab_h, gtab_h, dtab_h, wtab_h,
             item_out, g_out, d_out, w_out, *, iidx, gidx, didx, widx, ibuf,
             gbufs, dbufs, wbufs, gacc, dacc, wacc, sem):
        wid = lax.axis_index("s") * _NC + lax.axis_index("c")
        w_base = wid * tok_per_w

        def block(blk, carry):
            base = w_base + blk * _T
            sl_tok = pl.ds(base, _T)
            pltpu.sync_copy(item_ids_h.at[sl_tok], iidx)
            pltpu.sync_copy(gids_h.at[:, sl_tok], gidx)
            pltpu.sync_copy(dids_h.at[:, sl_tok], didx)
            pltpu.sync_copy(wids_h.at[:, sl_tok], widx)

            copies = [pltpu.async_copy(itab_h.at[iidx], ibuf, sem)]
            for s in range(5):
                copies.append(pltpu.async_copy(gtab_h.at[gidx.at[s]], gbufs[s], sem))
            for s in range(5):
                copies.append(pltpu.async_copy(dtab_h.at[didx.at[s]], dbufs[s], sem))
            for s in range(5):
                copies.append(pltpu.async_copy(wtab_h.at[widx.at[s]], wbufs[s], sem))
            for c in copies:
                c.wait()

            pltpu.sync_copy(ibuf, item_out.at[sl_tok])

            def tok(t, carry2):
                for c in range(_D // 16):
                    sl = pl.ds(c * 16, 16)
                    gacc[t, sl] = (gbufs[0][t, sl] + gbufs[1][t, sl]
                                   + gbufs[2][t, sl] + gbufs[3][t, sl]
                                   + gbufs[4][t, sl])
                    dacc[t, sl] = (dbufs[0][t, sl] + dbufs[1][t, sl]
                                   + dbufs[2][t, sl] + dbufs[3][t, sl]
                                   + dbufs[4][t, sl])
                    wacc[t, sl] = (wbufs[0][t, sl] + wbufs[1][t, sl]
                                   + wbufs[2][t, sl] + wbufs[3][t, sl]
                                   + wbufs[4][t, sl])
                return carry2

            lax.fori_loop(0, _T, tok, 0)

            pltpu.sync_copy(gacc, g_out.at[sl_tok])
            pltpu.sync_copy(dacc, d_out.at[sl_tok])
            pltpu.sync_copy(wacc, w_out.at[sl_tok])
            return carry

        lax.fori_loop(0, nblk, block, 0)

    return body(item_ids, gids, dids, wids, item_tab, g_tab, d_tab, w_tab)


def _tc_project(item_e, g_sum, d_sum, w_sum, w_eff, b2):
    BL = item_e.shape[0]
    BT = 1024
    grid = (BL // BT,)

    def mm(ie, g, d, w, wr, br, o):
        x = jnp.concatenate([ie[...], g[...], d[...], w[...]], axis=1)
        o[...] = jnp.dot(x, wr[...], preferred_element_type=jnp.float32) + br[...]

    return pl.pallas_call(
        mm,
        grid=grid,
        in_specs=[
            pl.BlockSpec((BT, _D), lambda i: (i, 0)),
            pl.BlockSpec((BT, _D), lambda i: (i, 0)),
            pl.BlockSpec((BT, _D), lambda i: (i, 0)),
            pl.BlockSpec((BT, _D), lambda i: (i, 0)),
            pl.BlockSpec((4 * _D, _D), lambda i: (0, 0)),
            pl.BlockSpec((1, _D), lambda i: (0, 0)),
        ],
        out_specs=pl.BlockSpec((BT, _D), lambda i: (i, 0)),
        out_shape=jax.ShapeDtypeStruct((BL, _D), jnp.float32),
    )(item_e, g_sum, d_sum, w_sum, w_eff, b2)


def kernel(item_ids, genre_ids, director_ids, writer_ids, item_table,
           genre_table, director_table, writer_table, W, b):
    B, L = item_ids.shape
    BL = B * L
    M = genre_ids.shape[-1]

    ii = item_ids.reshape(BL).astype(jnp.int32)
    gi = genre_ids.reshape(BL, M).T.astype(jnp.int32)
    di = director_ids.reshape(BL, M).T.astype(jnp.int32)
    wi = writer_ids.reshape(BL, M).T.astype(jnp.int32)

    item_e, g_sum, d_sum, w_sum = _sc_gather_pool(
        ii, gi, di, wi, item_table, genre_table, director_table, writer_table)

    w_eff = jnp.concatenate([W[:_D], W[_D:] * (1.0 / M)], axis=0)
    out = _tc_project(item_e, g_sum, d_sum, w_sum, w_eff, b.reshape(1, _D))
    return out.reshape(B, L, _D)
